# baseline (speedup 1.0000x reference)
import functools

import jax
import jax.numpy as jnp
from jax.experimental import pallas as pl
from jax.experimental.pallas import tpu as pltpu
from jax.experimental.pallas import tpu_sc as plsc

_B = 4096
_N = 200
_D = 64
_DP = 128
_NUM_IDX = _B * _N
_W = 128


def _sc_gather(table_padded, idx_a, idx_b):
    mesh = plsc.VectorSubcoreMesh(core_axis_name="c", subcore_axis_name="s")

    @functools.partial(
        pl.kernel,
        out_type=jax.ShapeDtypeStruct((_NUM_IDX, _DP), jnp.float32),
        mesh=mesh,
    )
    def k(table_hbm, ia_hbm, ib_hbm, out_hbm):
        def body(ia_vmem, ib_vmem, out_vmem):
            pltpu.sync_copy(table_hbm.at[ia_vmem.at[0]],
                            out_vmem.at[pl.ds(0, _W)])
            pltpu.sync_copy(table_hbm.at[ib_vmem.at[0]],
                            out_vmem.at[pl.ds(_W, _W)])

        pltpu.emit_pipeline(
            body,
            grid=(_NUM_IDX // (2 * _W),),
            in_specs=[pl.BlockSpec((1, _W), lambda i: (0, i)),
                      pl.BlockSpec((1, _W), lambda i: (0, i))],
            out_specs=[pl.BlockSpec((2 * _W, _DP), lambda i: (i, 0))],
            core_axis_name=("c", "s"),
            dimension_semantics=(pltpu.PARALLEL,),
        )(ia_hbm, ib_hbm, out_hbm)

    return k(table_padded, idx_a, idx_b)


def _tc_transpose(g3d):
    def body(in_ref, out_ref):
        out_ref[0] = in_ref[0][:, :_D].T

    return pl.pallas_call(
        body,
        grid=(_N,),
        in_specs=[pl.BlockSpec((1, _B, _DP), lambda i: (i, 0, 0))],
        out_specs=pl.BlockSpec((1, _D, _B), lambda i: (i, 0, 0)),
        out_shape=jax.ShapeDtypeStruct((_N, _D, _B), jnp.float32),
        compiler_params=pltpu.CompilerParams(
            dimension_semantics=("parallel",),
        ),
    )(g3d)


def kernel(atom_types, table):
    idx3 = atom_types.T.reshape(_NUM_IDX // (2 * _W), 2, _W).astype(jnp.int32)
    idx_a = idx3[:, 0].reshape(1, _NUM_IDX // 2)
    idx_b = idx3[:, 1].reshape(1, _NUM_IDX // 2)
    table_padded = jnp.pad(table, ((0, 0), (0, _DP - _D)))
    g2d = _sc_gather(table_padded, idx_a, idx_b)
    g3d = g2d.reshape(_N, _B, _DP)
    out = _tc_transpose(g3d)
    return out.transpose(2, 0, 1)


# speedup vs baseline: 1.0373x; 1.0373x over previous
import functools

import jax
import jax.numpy as jnp
from jax.experimental import pallas as pl
from jax.experimental.pallas import tpu as pltpu
from jax.experimental.pallas import tpu_sc as plsc

_B = 4096
_N = 200
_D = 64
_DP = 128
_NUM_IDX = _B * _N
_W = 128


def _sc_gather(table_padded, idx_flat):
    mesh = plsc.VectorSubcoreMesh(core_axis_name="c", subcore_axis_name="s")

    @functools.partial(
        pl.kernel,
        out_type=jax.ShapeDtypeStruct((_NUM_IDX, _DP), jnp.float32),
        mesh=mesh,
    )
    def k(table_hbm, idx_hbm, out_hbm):
        def body(idx_vmem, out_vmem):
            pltpu.sync_copy(table_hbm.at[idx_vmem.at[0]], out_vmem)

        pltpu.emit_pipeline(
            body,
            grid=(_NUM_IDX // _W,),
            in_specs=[pl.BlockSpec((1, _W), lambda i: (0, i))],
            out_specs=[pl.BlockSpec((_W, _DP), lambda i: (i, 0))],
            core_axis_name=("c", "s"),
            dimension_semantics=(pltpu.PARALLEL,),
        )(idx_hbm, out_hbm)

    return k(table_padded, idx_flat)


def _tc_transpose(g3d):
    def body(in_ref, out_ref):
        out_ref[0] = in_ref[0][:, :_D].T

    return pl.pallas_call(
        body,
        grid=(_N,),
        in_specs=[pl.BlockSpec((1, _B, _DP), lambda i: (i, 0, 0))],
        out_specs=pl.BlockSpec((1, _D, _B), lambda i: (i, 0, 0)),
        out_shape=jax.ShapeDtypeStruct((_N, _D, _B), jnp.float32),
        compiler_params=pltpu.CompilerParams(
            dimension_semantics=("parallel",),
        ),
    )(g3d)


def kernel(atom_types, table):
    idx = atom_types.T.reshape(1, _NUM_IDX).astype(jnp.int32)
    table_padded = jnp.pad(table, ((0, 0), (0, _DP - _D)))
    g2d = _sc_gather(table_padded, idx)
    g3d = g2d.reshape(_N, _B, _DP)
    out = _tc_transpose(g3d)
    return out.transpose(2, 0, 1)
